# baseline (speedup 1.0000x reference)
import functools
import math

import jax
import jax.numpy as jnp
from jax.experimental import pallas as pl
from jax.experimental.pallas import tpu as pltpu

_B, _N, _D, _H, _E, _C = 4, 2048, 128, 128, 32, 2
_GAMMA, _ZETA = -0.1, 1.1
_R = 512
_RH = _R // 2
_I = _N // _R
_ESCALE = math.sqrt(0.5 / math.sqrt(_E))


def _prologue_body(nf_ref, we_ref, be_ref, x2_ref, e_ref):
    x = nf_ref[0]
    nrm = jnp.sqrt(jnp.sum(x * x, axis=1, keepdims=True))
    xn = x / jnp.maximum(nrm, 1e-12)
    x2_ref[0, :, : _D] = xn.astype(jnp.bfloat16)
    lane = jax.lax.broadcasted_iota(jnp.int32, (_N, _D), 1)
    x2_ref[0, :, _D:] = jnp.where(lane == 0, 1.0, 0.0).astype(jnp.bfloat16)
    e_ref[0] = (jnp.tanh(
        jnp.dot(xn, we_ref[...], preferred_element_type=jnp.float32)
        + be_ref[...]) * _ESCALE).astype(jnp.bfloat16)


def _gate_agg(adj_half, ei, e_all, x2_all):
    z = jax.lax.dot_general(
        ei, e_all, (((1,), (1,)), ((), ())),
        preferred_element_type=jnp.float32).astype(jnp.bfloat16)
    ew = jnp.clip(jnp.tanh(z) * jnp.bfloat16(0.6) + jnp.bfloat16(0.5),
                  jnp.bfloat16(0.0), jnp.bfloat16(1.0))
    wadj = adj_half.astype(jnp.bfloat16) * ew
    return jnp.dot(wadj, x2_all, preferred_element_type=jnp.float32)


def _main_body(x2_ref, e_ref, adja_ref, adjb_ref, wc_ref, bc_ref, hc_ref):
    i = pl.program_id(1)
    e_all = e_ref[0]
    x2_all = x2_ref[0]
    for half, adj_ref in ((0, adja_ref), (1, adjb_ref)):
        ei = e_ref[0, pl.ds(i * _R + half * _RH, _RH), :]
        agg = _gate_agg(adj_ref[0], ei, e_all, x2_all)
        rs = agg[:, _D:_D + 1] + 1e-8
        h = agg[:, : _D] / rs
        hc = jnp.maximum(
            jnp.dot(h, wc_ref[...], preferred_element_type=jnp.float32)
            + bc_ref[...], 0.0)
        hc_ref[0, pl.ds(half * _RH, _RH), :] = hc.astype(jnp.bfloat16)


def _pool_body(h_ref, aa_ref, w1_ref, b1_ref, g_ref, bt_ref, w2_ref, b2_ref,
               out_ref):
    gs = []
    for b in range(_B):
        hb = h_ref[b]
        al = jnp.dot(hb, aa_ref[...], preferred_element_type=jnp.float32)
        m = jnp.max(al)
        p = jnp.exp(al - m)
        denom = jnp.sum(p)
        g = jax.lax.dot_general(
            p.astype(jnp.bfloat16), hb, (((0,), (0,)), ((), ())),
            preferred_element_type=jnp.float32) / denom
        gs.append(g)
    g = jnp.concatenate(gs, axis=0)
    y = jnp.maximum(
        jnp.dot(g, w1_ref[...], preferred_element_type=jnp.float32)
        + b1_ref[...], 0.0)
    mu = jnp.mean(y, axis=1, keepdims=True)
    var = jnp.mean((y - mu) * (y - mu), axis=1, keepdims=True)
    yn = (y - mu) / jnp.sqrt(var + 1e-5) * g_ref[...] + bt_ref[...]
    out_ref[...] = (jnp.dot(yn, w2_ref[...],
                            preferred_element_type=jnp.float32) + b2_ref[...])


@functools.partial(jax.jit, static_argnames=("interpret",))
def _run(node_feat, adjs, W_edge, b_edge, W_conv, b_conv, a_attn,
         W1, b1, g_ln, bt_ln, W2, b2, interpret=False):
    full = lambda shape: pl.BlockSpec(shape, lambda *_: (0,) * len(shape))

    x2, e = pl.pallas_call(
        _prologue_body,
        grid=(_B,),
        in_specs=[
            pl.BlockSpec((1, _N, _D), lambda b: (b, 0, 0)),
            full((_D, _E)), full((1, _E)),
        ],
        out_specs=[
            pl.BlockSpec((1, _N, 2 * _D), lambda b: (b, 0, 0)),
            pl.BlockSpec((1, _N, _E), lambda b: (b, 0, 0)),
        ],
        out_shape=[
            jax.ShapeDtypeStruct((_B, _N, 2 * _D), jnp.bfloat16),
            jax.ShapeDtypeStruct((_B, _N, _E), jnp.bfloat16),
        ],
        interpret=interpret,
    )(node_feat, W_edge, b_edge)

    hc = pl.pallas_call(
        _main_body,
        grid=(_B, _I),
        in_specs=[
            pl.BlockSpec((1, _N, 2 * _D), lambda b, i: (b, 0, 0)),
            pl.BlockSpec((1, _N, _E), lambda b, i: (b, 0, 0)),
            pl.BlockSpec((1, _RH, _N), lambda b, i: (b, 2 * i, 0)),
            pl.BlockSpec((1, _RH, _N), lambda b, i: (b, 2 * i + 1, 0)),
            full((_D, _H)), full((1, _H)),
        ],
        out_specs=pl.BlockSpec((1, _R, _H), lambda b, i: (b, i, 0)),
        out_shape=jax.ShapeDtypeStruct((_B, _N, _H), jnp.bfloat16),
        interpret=interpret,
    )(x2, e, adjs, adjs, W_conv, b_conv)

    return pl.pallas_call(
        _pool_body,
        grid=(1,),
        in_specs=[
            full((_B, _N, _H)),
            full((_H, 1)),
            full((_H, _H // 2)), full((1, _H // 2)),
            full((1, _H // 2)), full((1, _H // 2)),
            full((_H // 2, _C)), full((1, _C)),
        ],
        out_specs=full((_B, _C)),
        out_shape=jax.ShapeDtypeStruct((_B, _C), jnp.float32),
        interpret=interpret,
    )(hc, a_attn, W1, b1, g_ln, bt_ln, W2, b2)


def kernel(node_feat, labels, adjs, W_edge, b_edge, W_conv, b_conv, a_attn,
           W1, b1, g_ln, bt_ln, W2, b2, interpret=False):
    del labels
    return _run(node_feat, adjs,
                W_edge, b_edge.reshape(1, _E),
                W_conv, b_conv.reshape(1, _H),
                a_attn.reshape(_H, 1),
                W1, b1.reshape(1, _H // 2),
                g_ln.reshape(1, _H // 2), bt_ln.reshape(1, _H // 2),
                W2, b2.reshape(1, _C), interpret=interpret)


# speedup vs baseline: 1.2512x; 1.2512x over previous
import functools
import math

import jax
import jax.numpy as jnp
from jax.experimental import pallas as pl
from jax.experimental.pallas import tpu as pltpu

_B, _N, _D, _H, _E, _C = 4, 2048, 128, 128, 32, 2
_GAMMA, _ZETA = -0.1, 1.1
_R = 512
_I = _N // _R
_ESCALE = math.sqrt(0.5 / math.sqrt(_E))


def _main_body(nf_ref, adj_ref, we_ref, be_ref, wc_ref, bc_ref, aa_ref,
               g_ref, x2_s, e_s, h_s):
    b = pl.program_id(0)
    i = pl.program_id(1)

    @pl.when(i == 0)
    def _prologue():
        x = nf_ref[0]
        nrm2 = jnp.dot(x * x, jnp.ones((_D, 1), jnp.float32),
                       preferred_element_type=jnp.float32)
        xn = x / jnp.maximum(jnp.sqrt(nrm2), 1e-12)
        x2_s[:, : _D] = xn.astype(jnp.bfloat16)
        x2_s[:, _D:_D + 1] = jnp.ones((_N, 1), jnp.bfloat16)
        e_s[...] = (jnp.tanh(
            jnp.dot(xn, we_ref[...], preferred_element_type=jnp.float32)
            + be_ref[...]) * _ESCALE).astype(jnp.bfloat16)

    ei = e_s[pl.ds(i * _R, _R), :]
    z = jax.lax.dot_general(
        ei, e_s[...], (((1,), (1,)), ((), ())),
        preferred_element_type=jnp.float32).astype(jnp.bfloat16)
    ew = jnp.clip(jnp.tanh(z) * jnp.bfloat16(0.6) + jnp.bfloat16(0.5),
                  jnp.bfloat16(0.0), jnp.bfloat16(1.0))
    wadj = adj_ref[0].astype(jnp.bfloat16) * ew
    agg = jnp.dot(wadj, x2_s[...], preferred_element_type=jnp.float32)
    rs = agg[:, _D:_D + 1] + 1e-8
    h = agg[:, : _D] / rs
    hc = jnp.maximum(
        jnp.dot(h, wc_ref[...], preferred_element_type=jnp.float32)
        + bc_ref[...], 0.0)
    h_s[pl.ds(i * _R, _R), :] = hc

    @pl.when(i == _I - 1)
    def _epilogue():
        al = jax.lax.dot_general(
            aa_ref[...], h_s[...], (((1,), (1,)), ((), ())),
            preferred_element_type=jnp.float32)
        m = jnp.max(al)
        p = jnp.exp(al - m)
        denom = jnp.sum(p)
        g = jax.lax.dot_general(
            p, h_s[...], (((1,), (0,)), ((), ())),
            preferred_element_type=jnp.float32) / denom
        g_ref[pl.ds(b, 1), :] = g


def _head_body(g_ref, w1_ref, b1_ref, gl_ref, bt_ref, w2_ref, b2_ref,
               out_ref):
    y = jnp.maximum(
        jnp.dot(g_ref[...], w1_ref[...], preferred_element_type=jnp.float32)
        + b1_ref[...], 0.0)
    mu = jnp.mean(y, axis=1, keepdims=True)
    var = jnp.mean((y - mu) * (y - mu), axis=1, keepdims=True)
    yn = (y - mu) / jnp.sqrt(var + 1e-5) * gl_ref[...] + bt_ref[...]
    out_ref[...] = (jnp.dot(yn, w2_ref[...],
                            preferred_element_type=jnp.float32) + b2_ref[...])


@functools.partial(jax.jit, static_argnames=("interpret",))
def _run(node_feat, adjs, W_edge, b_edge, W_conv, b_conv, a_attn,
         W1, b1, g_ln, bt_ln, W2, b2, interpret=False):
    full = lambda shape: pl.BlockSpec(shape, lambda *_: (0,) * len(shape))

    g = pl.pallas_call(
        _main_body,
        grid=(_B, _I),
        in_specs=[
            pl.BlockSpec((1, _N, _D), lambda b, i: (b, 0, 0)),
            pl.BlockSpec((1, _R, _N), lambda b, i: (b, i, 0)),
            full((_D, _E)), full((1, _E)),
            full((_D, _H)), full((1, _H)),
            full((1, _H)),
        ],
        out_specs=pl.BlockSpec((_B, _H), lambda b, i: (0, 0)),
        out_shape=jax.ShapeDtypeStruct((_B, _H), jnp.float32),
        scratch_shapes=[
            pltpu.VMEM((_N, 2 * _D), jnp.bfloat16),
            pltpu.VMEM((_N, _E), jnp.bfloat16),
            pltpu.VMEM((_N, _H), jnp.float32),
        ],
        interpret=interpret,
    )(node_feat, adjs, W_edge, b_edge, W_conv, b_conv, a_attn)

    return pl.pallas_call(
        _head_body,
        grid=(1,),
        in_specs=[
            full((_B, _H)),
            full((_H, _H // 2)), full((1, _H // 2)),
            full((1, _H // 2)), full((1, _H // 2)),
            full((_H // 2, _C)), full((1, _C)),
        ],
        out_specs=full((_B, _C)),
        out_shape=jax.ShapeDtypeStruct((_B, _C), jnp.float32),
        interpret=interpret,
    )(g, W1, b1, g_ln, bt_ln, W2, b2)


def kernel(node_feat, labels, adjs, W_edge, b_edge, W_conv, b_conv, a_attn,
           W1, b1, g_ln, bt_ln, W2, b2, interpret=False):
    del labels
    return _run(node_feat, adjs,
                W_edge, b_edge.reshape(1, _E),
                W_conv, b_conv.reshape(1, _H),
                a_attn.reshape(1, _H),
                W1, b1.reshape(1, _H // 2),
                g_ln.reshape(1, _H // 2), bt_ln.reshape(1, _H // 2),
                W2, b2.reshape(1, _C), interpret=interpret)


# speedup vs baseline: 1.4200x; 1.1349x over previous
import functools
import math

import jax
import jax.numpy as jnp
from jax.experimental import pallas as pl
from jax.experimental.pallas import tpu as pltpu

_B, _N, _D, _H, _E, _C = 4, 2048, 128, 128, 32, 2
_GAMMA, _ZETA = -0.1, 1.1
_R = 1024
_I = _N // _R
_ESCALE = math.sqrt(0.5 / math.sqrt(_E))


def _main_body(nf_ref, adj_ref, we_ref, be_ref, wc_ref, bc_ref, aa_ref,
               w1_ref, b1_ref, gl_ref, bt_ref, w2_ref, b2_ref,
               out_ref, x2_s, e_s, h_s):
    b = pl.program_id(0)
    i = pl.program_id(1)

    @pl.when(i == 0)
    def _prologue():
        x = nf_ref[0]
        nrm2 = jnp.dot(x * x, jnp.ones((_D, 1), jnp.float32),
                       preferred_element_type=jnp.float32)
        xn = x / jnp.maximum(jnp.sqrt(nrm2), 1e-12)
        x2_s[:, : _D] = xn.astype(jnp.bfloat16)
        x2_s[:, _D:_D + 1] = jnp.ones((_N, 1), jnp.bfloat16)
        e_s[...] = (jnp.tanh(
            jnp.dot(xn, we_ref[...], preferred_element_type=jnp.float32)
            + be_ref[...]) * _ESCALE).astype(jnp.bfloat16)

    ei = e_s[pl.ds(i * _R, _R), :]
    z = jax.lax.dot_general(
        ei, e_s[...], (((1,), (1,)), ((), ())),
        preferred_element_type=jnp.float32).astype(jnp.bfloat16)
    ew = jnp.clip(jnp.tanh(z) * jnp.bfloat16(0.6) + jnp.bfloat16(0.5),
                  jnp.bfloat16(0.0), jnp.bfloat16(1.0))
    wadj = adj_ref[0].astype(jnp.bfloat16) * ew
    agg = jnp.dot(wadj, x2_s[...], preferred_element_type=jnp.float32)
    rs = agg[:, _D:_D + 1] + 1e-8
    h = agg[:, : _D] / rs
    hc = jnp.maximum(
        jnp.dot(h, wc_ref[...], preferred_element_type=jnp.float32)
        + bc_ref[...], 0.0)
    h_s[pl.ds(i * _R, _R), :] = hc

    @pl.when(i == _I - 1)
    def _epilogue():
        al = jax.lax.dot_general(
            aa_ref[...], h_s[...], (((1,), (1,)), ((), ())),
            preferred_element_type=jnp.float32)
        m = jnp.max(al)
        p = jnp.exp(al - m)
        denom = jnp.sum(p)
        g = jax.lax.dot_general(
            p, h_s[...], (((1,), (0,)), ((), ())),
            preferred_element_type=jnp.float32) / denom
        y = jnp.maximum(
            jnp.dot(g, w1_ref[...], preferred_element_type=jnp.float32)
            + b1_ref[...], 0.0)
        mu = jnp.mean(y, axis=1, keepdims=True)
        var = jnp.mean((y - mu) * (y - mu), axis=1, keepdims=True)
        yn = (y - mu) / jnp.sqrt(var + 1e-5) * gl_ref[...] + bt_ref[...]
        out_ref[0, 0, :] = (jnp.dot(
            yn, w2_ref[...], preferred_element_type=jnp.float32)
            + b2_ref[...])[0]


@functools.partial(jax.jit, static_argnames=("interpret",))
def _run(node_feat, adjs, W_edge, b_edge, W_conv, b_conv, a_attn,
         W1, b1, g_ln, bt_ln, W2, b2, interpret=False):
    full = lambda shape: pl.BlockSpec(shape, lambda *_: (0,) * len(shape))

    out = pl.pallas_call(
        _main_body,
        grid=(_B, _I),
        in_specs=[
            pl.BlockSpec((1, _N, _D), lambda b, i: (b, 0, 0)),
            pl.BlockSpec((1, _R, _N), lambda b, i: (b, i, 0)),
            full((_D, _E)), full((1, _E)),
            full((_D, _H)), full((1, _H)),
            full((1, _H)),
            full((_H, _H // 2)), full((1, _H // 2)),
            full((1, _H // 2)), full((1, _H // 2)),
            full((_H // 2, _C)), full((1, _C)),
        ],
        out_specs=pl.BlockSpec((1, 1, _C), lambda b, i: (b, 0, 0)),
        out_shape=jax.ShapeDtypeStruct((_B, 1, _C), jnp.float32),
        scratch_shapes=[
            pltpu.VMEM((_N, 2 * _D), jnp.bfloat16),
            pltpu.VMEM((_N, _E), jnp.bfloat16),
            pltpu.VMEM((_N, _H), jnp.float32),
        ],
        interpret=interpret,
    )(node_feat, adjs, W_edge, b_edge, W_conv, b_conv, a_attn,
      W1, b1, g_ln, bt_ln, W2, b2)
    return out.reshape(_B, _C)


def kernel(node_feat, labels, adjs, W_edge, b_edge, W_conv, b_conv, a_attn,
           W1, b1, g_ln, bt_ln, W2, b2, interpret=False):
    del labels
    return _run(node_feat, adjs,
                W_edge, b_edge.reshape(1, _E),
                W_conv, b_conv.reshape(1, _H),
                a_attn.reshape(1, _H),
                W1, b1.reshape(1, _H // 2),
                g_ln.reshape(1, _H // 2), bt_ln.reshape(1, _H // 2),
                W2, b2.reshape(1, _C), interpret=interpret)
